# Initial kernel scaffold; baseline (speedup 1.0000x reference)
import jax
import jax.numpy as jnp
from jax.experimental import pallas as pl


def kernel(input_, quotient_embed_weight, remainder_embed_weight):
    raise NotImplementedError("write your pallas kernel here")


# speedup vs baseline: 3.8851x; 3.8851x over previous
import functools

import jax
import jax.numpy as jnp
from jax import lax
from jax.experimental import pallas as pl
from jax.experimental.pallas import tpu as pltpu
from jax.experimental.pallas import tpu_sc as plsc

_NUM_BUCKETS = 1000
_B = 16384
_H = 50
_HP = 56
_D = 64
_L = 16
_NC = 2
_NS = 16
_NW = _NC * _NS
_BPW = _B // _NW
_PAIR = 2 * _HP
_PAIRS = _BPW // 2


def _tec_body(idx_hbm, qtab_hbm, rtab_hbm, out_hbm,
              q_v, r_v, qrows, rrows, out_v, sem0, sem1):
    wid = lax.axis_index("s") * _NC + lax.axis_index("c")
    slab = _BPW * _HP

    pltpu.sync_copy(idx_hbm.at[pl.ds(wid * slab, slab)], q_v)

    inv = jnp.float32(1.0 / _NUM_BUCKETS)

    def qr_body(i, _):
        x = q_v[pl.ds(i * _L, _L)]
        q0 = (x.astype(jnp.float32) * inv).astype(jnp.int32)
        r0 = x - q0 * _NUM_BUCKETS
        q = q0 + jnp.where(r0 >= _NUM_BUCKETS, 1, 0) - jnp.where(r0 < 0, 1, 0)
        r = x - q * _NUM_BUCKETS
        q_v[pl.ds(i * _L, _L)] = q
        r_v[pl.ds(i * _L, _L)] = r
        return 0
    lax.fori_loop(0, slab // _L, qr_body, 0, unroll=2)

    sems = (sem0, sem1)

    def fire(p, slot):
        off = p * _PAIR
        pltpu.async_copy(qtab_hbm.at[q_v.at[pl.ds(off, _PAIR)]],
                         qrows.at[slot], sems[slot])
        pltpu.async_copy(rtab_hbm.at[r_v.at[pl.ds(off, _PAIR)]],
                         rrows.at[slot], sems[slot])

    def drain(slot):
        pltpu.make_async_copy(qtab_hbm.at[pl.ds(0, _PAIR)],
                              qrows.at[slot], sems[slot]).wait()
        pltpu.make_async_copy(rtab_hbm.at[pl.ds(0, _PAIR)],
                              rrows.at[slot], sems[slot]).wait()

    zeros = jnp.zeros((_L,), jnp.float32)

    def reduce(p, slot):
        for k in range(2):
            def red_body(j, acc):
                rb = k * _HP + j
                return tuple(
                    acc[c]
                    + qrows[slot, rb, pl.ds(_L * c, _L)]
                    + rrows[slot, rb, pl.ds(_L * c, _L)]
                    for c in range(_D // _L))
            acc = lax.fori_loop(0, _H, red_body, (zeros,) * (_D // _L))
            b = 2 * p + k
            for c in range(_D // _L):
                out_v[b, pl.ds(_L * c, _L)] = acc[c]

    fire(0, 0)

    def pair_step(pp, _):
        p0 = 2 * pp
        fire(p0 + 1, 1)
        drain(0)
        reduce(p0, 0)

        @pl.when(pp < _PAIRS // 2 - 1)
        def _():
            fire(p0 + 2, 0)
        drain(1)
        reduce(p0 + 1, 1)
        return 0
    lax.fori_loop(0, _PAIRS // 2, pair_step, 0)

    pltpu.sync_copy(out_v, out_hbm.at[pl.ds(wid * _BPW, _BPW)])


_mesh = plsc.VectorSubcoreMesh(core_axis_name="c", subcore_axis_name="s")

_qr_bag = functools.partial(
    pl.kernel,
    mesh=_mesh,
    out_type=jax.ShapeDtypeStruct((_B, _D), jnp.float32),
    scratch_types=[
        pltpu.VMEM((_BPW * _HP,), jnp.int32),
        pltpu.VMEM((_BPW * _HP,), jnp.int32),
        pltpu.VMEM((2, _PAIR, _D), jnp.float32),
        pltpu.VMEM((2, _PAIR, _D), jnp.float32),
        pltpu.VMEM((_BPW, _D), jnp.float32),
        pltpu.SemaphoreType.DMA,
        pltpu.SemaphoreType.DMA,
    ],
    compiler_params=pltpu.CompilerParams(use_tc_tiling_on_sc=False),
)(_tec_body)


def kernel(input_, quotient_embed_weight, remainder_embed_weight):
    idx = input_.astype(jnp.int32)
    idx = jnp.pad(idx, ((0, 0), (0, _HP - _H)))
    return _qr_bag(idx.reshape(-1),
                   quotient_embed_weight, remainder_embed_weight)
